# baseline (speedup 1.0000x reference)
import functools

import jax
import jax.numpy as jnp
from jax import lax
from jax.experimental import pallas as pl
from jax.experimental.pallas import tpu as pltpu
from jax.experimental.pallas import tpu_sc as plsc

_B = 16
_P = 4096
_D = 192
_NC = 2
_NS = 16
_ROWS = _P // (_NC * _NS)


def _make_sc_kernel():
    mesh = plsc.VectorSubcoreMesh(core_axis_name="c", subcore_axis_name="s")

    @functools.partial(
        pl.kernel,
        mesh=mesh,
        compiler_params=pltpu.CompilerParams(use_tc_tiling_on_sc=True),
        out_type=jax.ShapeDtypeStruct((_B, _P, _D), jnp.float32),
        scratch_types=[
            pltpu.VMEM((_ROWS, _D), jnp.float32),
            pltpu.SemaphoreType.DMA,
        ],
    )
    def k(table_hbm, out_hbm, buf, sem):
        wid = lax.axis_index("s") * _NC + lax.axis_index("c")
        base = wid * _ROWS
        pltpu.sync_copy(table_hbm.at[pl.ds(base, _ROWS)], buf)
        handles = [
            pltpu.async_copy(buf, out_hbm.at[b, pl.ds(base, _ROWS)], sem)
            for b in range(_B)
        ]
        for h in handles:
            h.wait()

    return k


_sc_broadcast = _make_sc_kernel()


def kernel(x, pos_table):
    del x
    return _sc_broadcast(pos_table)


# speedup vs baseline: 1.1750x; 1.1750x over previous
import jax
import jax.numpy as jnp
from jax.experimental import pallas as pl
from jax.experimental.pallas import tpu as pltpu


def _body(t_ref, o_ref):
    o_ref[0] = t_ref[...]


def kernel(x, pos_table):
    B = x.shape[0]
    P, D = pos_table.shape
    return pl.pallas_call(
        _body,
        grid=(B,),
        in_specs=[pl.BlockSpec((P, D), lambda b: (0, 0))],
        out_specs=pl.BlockSpec((1, P, D), lambda b: (b, 0, 0)),
        out_shape=jax.ShapeDtypeStruct((B, P, D), jnp.float32),
        compiler_params=pltpu.CompilerParams(
            dimension_semantics=("arbitrary",),
        ),
    )(pos_table)
